# Initial kernel scaffold; baseline (speedup 1.0000x reference)
import functools

import jax
import jax.numpy as jnp
from jax import lax
from jax.experimental import pallas as pl
from jax.experimental.pallas import tpu as pltpu
from jax.experimental.pallas import tpu_sc as plsc

NC, NS, L = 2, 16, 16
NW = NC * NS

VECTORS, SUBDIM, CENTROIDS, CHUNK = 100000, 100, 256, 3
DIM = SUBDIM * CHUNK
ROW_I32 = 128
KSTRIDE = CENTROIDS + 1
SSTRIDE = KSTRIDE * CHUNK
CODES_PAD = 86352
NBLK = (SUBDIM + L - 1) // L

N_TOTAL = 4096 * 50
PER_W = N_TOTAL // NW
CHUNK_W = 32
NCHUNK = PER_W // CHUNK_W


def _sc_body(ids_hbm, tbl_hbm, codes_hbm, out_hbm, codes_v, ids_v, rows_v, out_v, sem):
    wid = lax.axis_index("s") * NC + lax.axis_index("c")
    base = wid * PER_W
    pltpu.sync_copy(codes_hbm, codes_v)
    pltpu.sync_copy(ids_hbm.at[pl.ds(base, PER_W)], ids_v)

    lanes = lax.iota(jnp.int32, L)

    def chunk_body(c, carry):
        idx_slice = ids_v.at[pl.ds(c * CHUNK_W, CHUNK_W)]
        pltpu.async_copy(tbl_hbm.at[idx_slice], rows_v, sem).wait()

        def word_body(w, wcarry):
            for b in range(NBLK):
                k16 = rows_v[w, pl.ds(L * b, L)]
                mask = (lanes < (SUBDIM - L * b)) if (L * b + L > SUBDIM) else None
                addr = (lanes + L * b) * SSTRIDE + k16 * CHUNK
                obase = w * DIM + (L * b + lanes) * CHUNK
                for j in range(CHUNK):
                    vals = plsc.load_gather(codes_v, [addr + j], mask=mask)
                    plsc.store_scatter(out_v, [obase + j], vals, mask=mask)
            return wcarry

        lax.fori_loop(0, CHUNK_W, word_body, 0)
        pltpu.sync_copy(
            out_v, out_hbm.at[pl.ds((base + c * CHUNK_W) * DIM, CHUNK_W * DIM)]
        )
        return carry

    lax.fori_loop(0, NCHUNK, chunk_body, 0)


_mesh = plsc.VectorSubcoreMesh(
    core_axis_name="c", subcore_axis_name="s", num_cores=NC, num_subcores=NS
)

_sc_call = functools.partial(
    pl.kernel,
    out_type=jax.ShapeDtypeStruct((N_TOTAL * DIM,), jnp.float32),
    mesh=_mesh,
    scratch_types=[
        pltpu.VMEM((CODES_PAD,), jnp.float32),
        pltpu.VMEM((PER_W,), jnp.int32),
        pltpu.VMEM((CHUNK_W, ROW_I32), jnp.int32),
        pltpu.VMEM((CHUNK_W * DIM,), jnp.float32),
        pltpu.SemaphoreType.DMA,
    ],
)(_sc_body)


@jax.jit
def kernel(input, indexes, codes):
    ids = input.reshape(-1)
    tbl = jnp.pad(indexes.astype(jnp.int32), ((0, 1), (0, ROW_I32 - SUBDIM)))
    tbl = tbl.at[VECTORS, :].set(CENTROIDS)
    codes_ext = jnp.pad(codes, ((0, 0), (0, 1), (0, 0))).reshape(-1)
    codes_ext = jnp.pad(codes_ext, (0, CODES_PAD - SUBDIM * SSTRIDE))
    out = _sc_call(ids, tbl, codes_ext)
    return out.reshape(input.shape + (DIM,))


# speedup vs baseline: 160.5774x; 160.5774x over previous
import functools

import jax
import jax.numpy as jnp
from jax import lax
from jax.experimental import pallas as pl
from jax.experimental.pallas import tpu as pltpu
from jax.experimental.pallas import tpu_sc as plsc

NC, NS, L = 2, 16, 16
NW = NC * NS

VECTORS, SUBDIM, CENTROIDS, CHUNK = 100000, 100, 256, 3
DIM = SUBDIM * CHUNK
ROW_I32 = 128
KSTRIDE = CENTROIDS + 1
SSTRIDE = KSTRIDE * CHUNK
CODES_PAD = 86352
NBLK = (SUBDIM + L - 1) // L

N_TOTAL = 4096 * 50
PER_W = N_TOTAL // NW
CHUNK_W = 32
NCHUNK = PER_W // CHUNK_W


def _sc_body(ids_hbm, tbl_hbm, codes_hbm, out_hbm, codes_v, ids_v, rows_v, out_v, sem):
    wid = lax.axis_index("s") * NC + lax.axis_index("c")
    base = wid * PER_W
    pltpu.sync_copy(codes_hbm, codes_v)
    pltpu.sync_copy(ids_hbm.at[pl.ds(wid * NCHUNK, NCHUNK)], ids_v)

    lanes = lax.iota(jnp.int32, L)

    def chunk_body(c, carry):
        pltpu.async_copy(tbl_hbm.at[ids_v.at[c]], rows_v, sem).wait()

        def word_body(w, wcarry):
            for b in range(NBLK):
                k16 = rows_v[w, pl.ds(L * b, L)]
                mask = (lanes < (SUBDIM - L * b)) if (L * b + L > SUBDIM) else None
                addr = (lanes + L * b) * SSTRIDE + k16 * CHUNK
                obase = w * DIM + (L * b + lanes) * CHUNK
                for j in range(CHUNK):
                    vals = plsc.load_gather(codes_v, [addr + j], mask=mask)
                    plsc.store_scatter(out_v, [obase + j], vals, mask=mask)
            return wcarry

        lax.fori_loop(0, CHUNK_W, word_body, 0)
        pltpu.sync_copy(
            out_v.at[pl.ds(0, CHUNK_W * DIM)],
            out_hbm.at[pl.ds((base + c * CHUNK_W) * DIM, CHUNK_W * DIM)],
        )
        return carry

    lax.fori_loop(0, NCHUNK, chunk_body, 0)


_mesh = plsc.VectorSubcoreMesh(
    core_axis_name="c", subcore_axis_name="s", num_cores=NC, num_subcores=NS
)

_sc_call = functools.partial(
    pl.kernel,
    out_type=jax.ShapeDtypeStruct((N_TOTAL * DIM,), jnp.float32),
    mesh=_mesh,
    compiler_params=pltpu.CompilerParams(needs_layout_passes=False),
    scratch_types=[
        pltpu.VMEM((CODES_PAD,), jnp.float32),
        pltpu.VMEM((NCHUNK, CHUNK_W), jnp.int32),
        pltpu.VMEM((CHUNK_W, ROW_I32), jnp.int32),
        pltpu.VMEM((CHUNK_W * DIM + 48,), jnp.float32),
        pltpu.SemaphoreType.DMA,
    ],
)(_sc_body)


@jax.jit
def kernel(input, indexes, codes):
    ids = input.reshape(N_TOTAL // CHUNK_W, CHUNK_W)
    tbl = jnp.pad(indexes.astype(jnp.int32), ((0, 1), (0, ROW_I32 - SUBDIM)))
    tbl = tbl.at[VECTORS, :].set(CENTROIDS)
    codes_ext = jnp.pad(codes, ((0, 0), (0, 1), (0, 0))).reshape(-1)
    codes_ext = jnp.pad(codes_ext, (0, CODES_PAD - SUBDIM * SSTRIDE))
    out = _sc_call(ids, tbl, codes_ext)
    return out.reshape(input.shape + (DIM,))


# speedup vs baseline: 186.2154x; 1.1597x over previous
import functools

import jax
import jax.numpy as jnp
from jax import lax
from jax.experimental import pallas as pl
from jax.experimental.pallas import tpu as pltpu
from jax.experimental.pallas import tpu_sc as plsc

NC, NS, L = 2, 16, 16
NW = NC * NS

VECTORS, SUBDIM, CENTROIDS, CHUNK = 100000, 100, 256, 3
DIM = SUBDIM * CHUNK
ROW_I32 = 128
KSTRIDE = CENTROIDS + 1
SSTRIDE = KSTRIDE * CHUNK
CODES_PAD = 77104
NBLK = (SUBDIM + L - 1) // L

N_TOTAL = 4096 * 50
PER_W = N_TOTAL // NW
CHUNK_W = 32
NCHUNK = PER_W // CHUNK_W


def _sc_body(
    ids_hbm, tbl_hbm, codes_hbm, out_hbm,
    codes_v, ids_v, rows_v0, rows_v1, out_v0, out_v1,
    sem_r0, sem_r1, sem_o0, sem_o1,
):
    wid = lax.axis_index("s") * NC + lax.axis_index("c")
    base = wid * PER_W
    pltpu.sync_copy(codes_hbm, codes_v)
    pltpu.sync_copy(ids_hbm.at[pl.ds(wid * NCHUNK, NCHUNK)], ids_v)

    lanes = lax.iota(jnp.int32, L)
    sem_r = (sem_r0, sem_r1)
    sem_o = (sem_o0, sem_o1)
    rows_v = (rows_v0, rows_v1)
    out_v = (out_v0, out_v1)

    def row_copy(c, b):
        return pltpu.make_async_copy(tbl_hbm.at[ids_v.at[c]], rows_v[b], sem_r[b])

    def out_copy(c, b):
        return pltpu.make_async_copy(
            out_v[b].at[pl.ds(0, CHUNK_W * DIM)],
            out_hbm.at[pl.ds((base + c * CHUNK_W) * DIM, CHUNK_W * DIM)],
            sem_o[b],
        )

    row_copy(0, 0).start()

    def outer_body(co, carry):
        for b in range(2):
            c = 2 * co + b
            row_copy(c, b).wait()

            @pl.when(c < NCHUNK - 1)
            def _():
                row_copy(c + 1, 1 - b).start()

            @pl.when(co > 0)
            def _():
                out_copy(c - 2, b).wait()

            def word_body(w, wcarry):
                for blk in range(NBLK):
                    k16 = rows_v[b][w, pl.ds(L * blk, L)]
                    mask = (
                        (lanes < (SUBDIM - L * blk))
                        if (L * blk + L > SUBDIM)
                        else None
                    )
                    addr = (lanes + L * blk) * SSTRIDE + k16 * CHUNK
                    obase = w * DIM + (L * blk + lanes) * CHUNK
                    for j in range(CHUNK):
                        vals = plsc.load_gather(codes_v, [addr + j], mask=mask)
                        plsc.store_scatter(out_v[b], [obase + j], vals, mask=mask)
                return wcarry

            lax.fori_loop(0, CHUNK_W, word_body, 0)
            out_copy(c, b).start()
        return carry

    lax.fori_loop(0, NCHUNK // 2, outer_body, 0)
    for b in range(2):
        out_copy(NCHUNK - 2 + b, b).wait()


_mesh = plsc.VectorSubcoreMesh(
    core_axis_name="c", subcore_axis_name="s", num_cores=NC, num_subcores=NS
)

_sc_call = functools.partial(
    pl.kernel,
    out_type=jax.ShapeDtypeStruct((N_TOTAL * DIM,), jnp.float32),
    mesh=_mesh,
    compiler_params=pltpu.CompilerParams(needs_layout_passes=False),
    scratch_types=[
        pltpu.VMEM((CODES_PAD,), jnp.float32),
        pltpu.VMEM((NCHUNK, CHUNK_W), jnp.int32),
        pltpu.VMEM((CHUNK_W, ROW_I32), jnp.int32),
        pltpu.VMEM((CHUNK_W, ROW_I32), jnp.int32),
        pltpu.VMEM((CHUNK_W * DIM + 48,), jnp.float32),
        pltpu.VMEM((CHUNK_W * DIM + 48,), jnp.float32),
        pltpu.SemaphoreType.DMA,
        pltpu.SemaphoreType.DMA,
        pltpu.SemaphoreType.DMA,
        pltpu.SemaphoreType.DMA,
    ],
)(_sc_body)


@jax.jit
def kernel(input, indexes, codes):
    ids = input.reshape(N_TOTAL // CHUNK_W, CHUNK_W)
    tbl = jnp.pad(indexes.astype(jnp.int32), ((0, 1), (0, ROW_I32 - SUBDIM)))
    tbl = tbl.at[VECTORS, :].set(CENTROIDS)
    codes_ext = jnp.pad(codes, ((0, 0), (0, 1), (0, 0))).reshape(-1)
    codes_ext = jnp.pad(codes_ext, (0, CODES_PAD - SUBDIM * SSTRIDE))
    out = _sc_call(ids, tbl, codes_ext)
    return out.reshape(input.shape + (DIM,))


# speedup vs baseline: 304.8530x; 1.6371x over previous
import functools

import jax
import jax.numpy as jnp
from jax import lax
from jax.experimental import pallas as pl
from jax.experimental.pallas import tpu as pltpu
from jax.experimental.pallas import tpu_sc as plsc

NC, NS, L = 2, 16, 16
NW = NC * NS

VECTORS, SUBDIM, CENTROIDS, CHUNK = 100000, 100, 256, 3
DIM = SUBDIM * CHUNK
ROW_I32 = 128
KSTRIDE = CENTROIDS + 1
SSTRIDE = KSTRIDE * CHUNK
CODES_PAD = 77104
NBLK = (SUBDIM + L - 1) // L

N_TOTAL = 4096 * 50
PER_W = N_TOTAL // NW
CHUNK_W = 32
NCHUNK = PER_W // CHUNK_W


def _sc_body(
    ids_hbm, tbl_hbm, codes_hbm, out_hbm,
    codes_v, ids_v, rows_v0, rows_v1, out_v0, out_v1,
    sem_r0, sem_r1, sem_o0, sem_o1,
):
    wid = lax.axis_index("s") * NC + lax.axis_index("c")
    base = wid * PER_W
    pltpu.sync_copy(codes_hbm, codes_v)
    pltpu.sync_copy(ids_hbm.at[pl.ds(wid * NCHUNK, NCHUNK)], ids_v)

    lanes = lax.iota(jnp.int32, L)
    sem_r = (sem_r0, sem_r1)
    sem_o = (sem_o0, sem_o1)
    rows_v = (rows_v0, rows_v1)
    out_v = (out_v0, out_v1)

    def row_copy(c, b):
        return pltpu.make_async_copy(tbl_hbm.at[ids_v.at[c]], rows_v[b], sem_r[b])

    def out_copy(c, b):
        return pltpu.make_async_copy(
            out_v[b].at[pl.ds(0, CHUNK_W * DIM)],
            out_hbm.at[pl.ds((base + c * CHUNK_W) * DIM, CHUNK_W * DIM)],
            sem_o[b],
        )

    row_copy(0, 0).start()

    def outer_body(co, carry):
        for b in range(2):
            c = 2 * co + b
            row_copy(c, b).wait()

            @pl.when(c < NCHUNK - 1)
            def _():
                row_copy(c + 1, 1 - b).start()

            @pl.when(co > 0)
            def _():
                out_copy(c - 2, b).wait()

            @plsc.parallel_loop(0, CHUNK_W, 1, unroll=2)
            def word_body(w):
                for blk in range(NBLK):
                    k16 = rows_v[b][w, pl.ds(L * blk, L)]
                    mask = (
                        (lanes < (SUBDIM - L * blk))
                        if (L * blk + L > SUBDIM)
                        else None
                    )
                    addr = (lanes + L * blk) * SSTRIDE + k16 * CHUNK
                    obase = w * DIM + (L * blk + lanes) * CHUNK
                    for j in range(CHUNK):
                        vals = plsc.load_gather(codes_v, [addr + j], mask=mask)
                        plsc.store_scatter(out_v[b], [obase + j], vals, mask=mask)
            out_copy(c, b).start()
        return carry

    lax.fori_loop(0, NCHUNK // 2, outer_body, 0)
    for b in range(2):
        out_copy(NCHUNK - 2 + b, b).wait()


_mesh = plsc.VectorSubcoreMesh(
    core_axis_name="c", subcore_axis_name="s", num_cores=NC, num_subcores=NS
)

_sc_call = functools.partial(
    pl.kernel,
    out_type=jax.ShapeDtypeStruct((N_TOTAL * DIM,), jnp.float32),
    mesh=_mesh,
    compiler_params=pltpu.CompilerParams(needs_layout_passes=False),
    scratch_types=[
        pltpu.VMEM((CODES_PAD,), jnp.float32),
        pltpu.VMEM((NCHUNK, CHUNK_W), jnp.int32),
        pltpu.VMEM((CHUNK_W, ROW_I32), jnp.int32),
        pltpu.VMEM((CHUNK_W, ROW_I32), jnp.int32),
        pltpu.VMEM((CHUNK_W * DIM + 48,), jnp.float32),
        pltpu.VMEM((CHUNK_W * DIM + 48,), jnp.float32),
        pltpu.SemaphoreType.DMA,
        pltpu.SemaphoreType.DMA,
        pltpu.SemaphoreType.DMA,
        pltpu.SemaphoreType.DMA,
    ],
)(_sc_body)


@jax.jit
def kernel(input, indexes, codes):
    ids = input.reshape(N_TOTAL // CHUNK_W, CHUNK_W)
    tbl = jnp.pad(indexes.astype(jnp.int32), ((0, 1), (0, ROW_I32 - SUBDIM)))
    tbl = tbl.at[VECTORS, :].set(CENTROIDS)
    codes_ext = jnp.pad(codes, ((0, 0), (0, 1), (0, 0))).reshape(-1)
    codes_ext = jnp.pad(codes_ext, (0, CODES_PAD - SUBDIM * SSTRIDE))
    out = _sc_call(ids, tbl, codes_ext)
    return out.reshape(input.shape + (DIM,))
